# Initial kernel scaffold; baseline (speedup 1.0000x reference)
import jax
import jax.numpy as jnp
from jax.experimental import pallas as pl


def kernel(idxes, pe):
    raise NotImplementedError("write your pallas kernel here")


# speedup vs baseline: 4.1340x; 4.1340x over previous
import functools

import jax
import jax.numpy as jnp
from jax import lax
from jax.experimental import pallas as pl
from jax.experimental.pallas import tpu as pltpu
from jax.experimental.pallas import tpu_sc as plsc

_D = 64
_B = 4096 * 200

_info = plsc.get_sparse_core_info()
_NC = _info.num_cores
_NS = _info.num_subcores
_NW = _NC * _NS
_B_PER_W = _B // _NW
_CHUNK = 1024
_N_CHUNKS = _B_PER_W // _CHUNK

_mesh = plsc.VectorSubcoreMesh(core_axis_name="c", subcore_axis_name="s")


@functools.partial(
    pl.kernel,
    mesh=_mesh,
    out_type=jax.ShapeDtypeStruct((_B, _D), jnp.float32),
    compiler_params=pltpu.CompilerParams(use_tc_tiling_on_sc=False),
    scratch_types=[
        pltpu.VMEM((_CHUNK,), jnp.int32),
        pltpu.VMEM((_CHUNK, _D), jnp.float32),
        pltpu.SemaphoreType.DMA,
    ],
)
def _gather_kernel(idx_hbm, table_hbm, out_hbm, idx_v, rows_v, sem):
    wid = lax.axis_index("s") * _NC + lax.axis_index("c")
    base = pl.multiple_of(wid * _B_PER_W, 8)

    def body(i, carry):
        off = pl.multiple_of(base + i * _CHUNK, 8)
        pltpu.sync_copy(idx_hbm.at[pl.ds(off, _CHUNK)], idx_v)
        pltpu.async_copy(table_hbm.at[idx_v], rows_v, sem).wait()
        pltpu.sync_copy(rows_v, out_hbm.at[pl.ds(off, _CHUNK)])
        return carry

    lax.fori_loop(0, _N_CHUNKS, body, 0)


def kernel(idxes, pe):
    flat = idxes.reshape(-1).astype(jnp.int32)
    out = _gather_kernel(flat, pe)
    return out.reshape(idxes.shape[0], idxes.shape[1], _D)


# speedup vs baseline: 4.5618x; 1.1035x over previous
import functools

import jax
import jax.numpy as jnp
from jax import lax
from jax.experimental import pallas as pl
from jax.experimental.pallas import tpu as pltpu
from jax.experimental.pallas import tpu_sc as plsc

_D = 64
_DP = 128
_B = 4096 * 200

_info = plsc.get_sparse_core_info()
_NC = _info.num_cores
_NS = _info.num_subcores
_NW = _NC * _NS
_B_PER_W = _B // _NW
_CHUNK = 1024
_N_CHUNKS = _B_PER_W // _CHUNK

_mesh = plsc.VectorSubcoreMesh(core_axis_name="c", subcore_axis_name="s")


@functools.partial(
    pl.kernel,
    mesh=_mesh,
    out_type=jax.ShapeDtypeStruct((_B, _D), jnp.float32),
    compiler_params=pltpu.CompilerParams(use_tc_tiling_on_sc=False),
    scratch_types=[
        pltpu.VMEM((_CHUNK,), jnp.int32),
        pltpu.VMEM((_CHUNK, _D), jnp.float32),
        pltpu.SemaphoreType.DMA,
    ],
)
def _gather_kernel(idx_hbm, table_hbm, out_hbm, idx_v, rows_v, sem):
    wid = lax.axis_index("s") * _NC + lax.axis_index("c")
    base = pl.multiple_of(wid * _B_PER_W, 8)

    def body(i, carry):
        off = pl.multiple_of(base + i * _CHUNK, 8)
        pltpu.sync_copy(idx_hbm.at[pl.ds(off, _CHUNK)], idx_v)
        pltpu.async_copy(table_hbm.at[idx_v], rows_v, sem).wait()
        pltpu.sync_copy(rows_v, out_hbm.at[pl.ds(off, _CHUNK)])
        return carry

    lax.fori_loop(0, _N_CHUNKS, body, 0)


_BB = 512
_BT = 512
_TK = 200 * _D


def _tr_body(i_ref, o_ref):
    o_ref[...] = i_ref[...].T


_transpose = pl.pallas_call(
    _tr_body,
    grid=(4096 // _BB, _TK // _BT),
    in_specs=[pl.BlockSpec((_BB, _BT), lambda i, j: (i, j))],
    out_specs=pl.BlockSpec((_BT, _BB), lambda i, j: (j, i)),
    out_shape=jax.ShapeDtypeStruct((_TK, 4096), jnp.float32),
)


def kernel(idxes, pe):
    flat = idxes.reshape(-1).astype(jnp.int32)
    lin = _gather_kernel(flat, pe)
    t2 = _transpose(lin.reshape(4096, _TK))
    return t2.reshape(200, _D, 4096).transpose(2, 0, 1)


# speedup vs baseline: 7.9742x; 1.7480x over previous
import functools

import jax
import jax.numpy as jnp
from jax import lax
from jax.experimental import pallas as pl
from jax.experimental.pallas import tpu as pltpu
from jax.experimental.pallas import tpu_sc as plsc

_D = 64
_B = 4096 * 200
_TK = 200 * _D

_info = plsc.get_sparse_core_info()
_NC = _info.num_cores
_NS = _info.num_subcores
_NW = _NC * _NS
_B_PER_W = _B // _NW
_CHUNK = 1024
_N_CHUNKS = _B_PER_W // _CHUNK

_mesh = plsc.VectorSubcoreMesh(core_axis_name="c", subcore_axis_name="s")


@functools.partial(
    pl.kernel,
    mesh=_mesh,
    out_type=jax.ShapeDtypeStruct((_B, _D), jnp.float32),
    compiler_params=pltpu.CompilerParams(use_tc_tiling_on_sc=False),
    scratch_types=[
        pltpu.VMEM((_CHUNK,), jnp.int32),
        pltpu.VMEM((_CHUNK, _D), jnp.float32),
        pltpu.SemaphoreType.DMA,
    ],
)
def _gather_kernel(idx_hbm, table_hbm, out_hbm, idx_v, rows_v, sem):
    wid = lax.axis_index("s") * _NC + lax.axis_index("c")
    base = pl.multiple_of(wid * _B_PER_W, 8)

    def body(i, carry):
        off = pl.multiple_of(base + i * _CHUNK, 8)
        pltpu.sync_copy(idx_hbm.at[pl.ds(off, _CHUNK)], idx_v)
        pltpu.async_copy(table_hbm.at[idx_v], rows_v, sem).wait()
        pltpu.sync_copy(rows_v, out_hbm.at[pl.ds(off, _CHUNK)])
        return carry

    lax.fori_loop(0, _N_CHUNKS, body, 0)


_BB = 128
_NSTEPS = 4096 // _BB


def _tr_body(lin_hbm, o_ref, buf, sems):
    step = pl.program_id(0)
    slot = lax.rem(step, 2)
    nslot = lax.rem(step + 1, 2)

    def issue(dst_slot, blk):
        for r in range(_BB):
            pltpu.make_async_copy(
                lin_hbm.at[pl.ds((blk * _BB + r) * _TK, _TK)],
                buf.at[dst_slot, r],
                sems.at[dst_slot],
            ).start()

    def drain(dst_slot, blk):
        for r in range(_BB):
            pltpu.make_async_copy(
                lin_hbm.at[pl.ds((blk * _BB + r) * _TK, _TK)],
                buf.at[dst_slot, r],
                sems.at[dst_slot],
            ).wait()

    @pl.when(step == 0)
    def _():
        issue(slot, step)

    @pl.when(step + 1 < _NSTEPS)
    def _():
        issue(nslot, step + 1)

    drain(slot, step)
    for j in range(_TK // 128):
        o_ref[pl.ds(j * 128, 128), :] = buf[slot, :, pl.ds(j * 128, 128)].T


_transpose = pl.pallas_call(
    _tr_body,
    grid=(_NSTEPS,),
    in_specs=[pl.BlockSpec(memory_space=pl.ANY)],
    out_specs=pl.BlockSpec((_TK, _BB), lambda i: (0, i)),
    out_shape=jax.ShapeDtypeStruct((_TK, 4096), jnp.float32),
    scratch_shapes=[
        pltpu.VMEM((2, _BB, _TK), jnp.float32),
        pltpu.SemaphoreType.DMA((2,)),
    ],
)


def kernel(idxes, pe):
    flat = idxes.reshape(-1).astype(jnp.int32)
    lin = _gather_kernel(flat, pe)
    t2 = _transpose(lin.reshape(-1))
    return t2.reshape(200, _D, 4096).transpose(2, 0, 1)
